# baseline (speedup 1.0000x reference)
import functools

import jax
import jax.numpy as jnp
from jax import lax
from jax.experimental import pallas as pl
from jax.experimental.pallas import tpu as pltpu
from jax.experimental.pallas import tpu_sc as plsc

BATCH = 4096
SEQ = 50
VOCAB = 1000
D = 1000
NSHARD = 8
TAIL = D - 128 * (NSHARD - 1)

NUM_WORKERS = 32
NB = BATCH // NUM_WORKERS
NBUF = 2

NCHUNK = 2
SEQ_C = SEQ // NCHUNK
SEQ_CP = (SEQ_C + 7) // 8 * 8

_MESH = plsc.VectorSubcoreMesh(core_axis_name="c", subcore_axis_name="s")


@functools.partial(
    pl.kernel,
    out_type=jax.ShapeDtypeStruct((BATCH, SEQ_C, D), jnp.float32),
    mesh=_MESH,
    scratch_types=[
        pltpu.VMEM((NB * SEQ_CP,), jnp.int32),
        pltpu.VMEM((NBUF, NSHARD, SEQ_C, 128), jnp.float32),
        pltpu.VMEM((SEQ_C, TAIL), jnp.float32),
        pltpu.SemaphoreType.DMA((NBUF,)),
        pltpu.SemaphoreType.DMA((NBUF,)),
        pltpu.SemaphoreType.DMA,
    ],
    compiler_params=pltpu.CompilerParams(use_tc_tiling_on_sc=True),
)
def _gather_chunk(xf_hbm, *refs):
    shards = refs[:NSHARD]
    out_hbm = refs[NSHARD]
    idx_v, rows, tail_v, sem_g, sem_w, sem_t = refs[NSHARD + 1:]

    wid = lax.axis_index("s") * 2 + lax.axis_index("c")
    base = wid * NB

    def idx_slice(k):
        return idx_v.at[pl.ds(pl.multiple_of(k * SEQ_CP, 8), SEQ_C)]

    def gather_copy(k, m, c):
        return pltpu.make_async_copy(shards[c].at[idx_slice(k)],
                                     rows.at[m, c], sem_g.at[m])

    def shard_write(bb, m, c):
        return pltpu.make_async_copy(
            rows.at[m, c], out_hbm.at[bb].at[:, pl.ds(c * 128, 128)],
            sem_w.at[m])

    def tail_write(bb):
        return pltpu.make_async_copy(
            tail_v, out_hbm.at[bb].at[:, pl.ds(128 * (NSHARD - 1), TAIL)],
            sem_t)

    def tail_compact(m):
        def row(s, cr):
            for off in (0, 16, 32, 48, 64, 80, TAIL - 16):
                tail_v[s, pl.ds(off, 16)] = rows[m, NSHARD - 1, s,
                                                 pl.ds(off, 16)]
            return cr

        lax.fori_loop(0, SEQ_C, row, 0)

    pltpu.sync_copy(xf_hbm.at[pl.ds(base * SEQ_CP, NB * SEQ_CP)], idx_v)

    for m in range(NBUF):
        for c in range(NSHARD):
            gather_copy(m, m, c).start()

    def pair(g, cr):
        for m in range(NBUF):
            k = g + m
            bb = base + k
            for c in range(NSHARD):
                gather_copy(k, m, c).wait()
            for c in range(NSHARD - 1):
                shard_write(bb, m, c).start()

            @pl.when(k >= 1)
            def _():
                tail_write(bb - 1).wait()

            tail_compact(m)
            tail_write(bb).start()

            @pl.when(k + NBUF < NB)
            def _():
                for c in range(NSHARD - 1):
                    shard_write(bb, m, c).wait()
                for c in range(NSHARD):
                    gather_copy(k + NBUF, m, c).start()

        return cr

    lax.fori_loop(0, NB // NBUF, lambda i, cr: pair(i * NBUF, cr), 0)

    for m in range(NBUF):
        for c in range(NSHARD - 1):
            shard_write(base + NB - NBUF + m, m, c).wait()
    tail_write(base + NB - 1).wait()


def kernel(x, table):
    tp = jnp.pad(table, ((0, 0), (0, NSHARD * 128 - D)))
    shards = tuple(tp[:, c * 128:(c + 1) * 128] for c in range(NSHARD))
    chunks = []
    for q in range(NCHUNK):
        xq = x[:, q * SEQ_C:(q + 1) * SEQ_C]
        xf = jnp.pad(xq, ((0, 0), (0, SEQ_CP - SEQ_C))).reshape(-1)
        chunks.append(_gather_chunk(xf, *shards))
    return jnp.concatenate(chunks, axis=1)
